# baseline (speedup 1.0000x reference)
import jax
import jax.numpy as jnp
from jax import lax
from jax.experimental import pallas as pl
from jax.experimental.pallas import tpu as pltpu
from jax.experimental.pallas import tpu_sc as plsc

NC = 2
NS = 16
NW = NC * NS
LANE = 128
CHUNK = 8
F = 4

NPAD = 102400
RPW = 784
EPAD = NW * RPW * LANE
SLICE = NPAD // NS
NODES_W = NPAD // NW
ROWS_W = NODES_W * F // LANE
PACK = NPAD * F // LANE


def _sc_mesh():
    return plsc.VectorSubcoreMesh(core_axis_name="c", subcore_axis_name="s")


_SC_PARAMS = pltpu.CompilerParams(use_tc_tiling_on_sc=False)
_SC_EW_PARAMS = pltpu.CompilerParams(use_tc_tiling_on_sc=False,
                                     needs_layout_passes=False)


def _worker_ids():
    cid = lax.axis_index("c")
    sid = lax.axis_index("s")
    return cid, sid, sid * NC + cid


def _zero_acc(zeros_h, acc, sid):
    sl = pl.ds(sid * SLICE, SLICE)
    pltpu.sync_copy(zeros_h.at[sl], acc.at[sl])


def _copy_out(acc, out_h, cid, sid):
    sl = pl.ds(sid * SLICE, SLICE)
    pltpu.sync_copy(acc.at[sl], out_h.at[cid].at[sl])


def _rsqrt16(v):
    i = plsc.bitcast(v, jnp.int32)
    i = jnp.int32(0x5F3759DF) - lax.shift_right_arithmetic(i, 1)
    y = plsc.bitcast(i, jnp.float32)
    for _ in range(3):
        y = y * (1.5 - 0.5 * v * y * y)
    return y


def _deg_kernel(ones_h, dst_h, zeros_h, out_h, acc, idx_d, ones_v, sem_i, sem_s):
    cid, sid, wid = _worker_ids()
    _zero_acc(zeros_h, acc, sid)
    pltpu.sync_copy(ones_h, ones_v)
    plsc.subcore_barrier()

    @pl.loop(0, RPW, step=CHUNK)
    def _(i):
        pltpu.async_copy(dst_h.at[wid].at[pl.ds(i, CHUNK)], idx_d, sem_i).wait()
        adds = [
            pltpu.async_copy(ones_v, acc.at[idx_d.at[j]], sem_s, add=True)
            for j in range(CHUNK)
        ]
        for a in adds:
            a.wait()

    plsc.subcore_barrier()
    _copy_out(acc, out_h, cid, sid)


def _agg_kernel(table_h, src_h, dst_h, zeros_h, out_h,
                acc, idx_s, idx_d, rows, sem_i, sem_g, sem_s):
    cid, sid, wid = _worker_ids()
    _zero_acc(zeros_h, acc, sid)
    plsc.subcore_barrier()

    @pl.loop(0, RPW, step=CHUNK)
    def _(i):
        ci = pltpu.async_copy(src_h.at[wid].at[pl.ds(i, CHUNK)], idx_s, sem_i)
        cj = pltpu.async_copy(dst_h.at[wid].at[pl.ds(i, CHUNK)], idx_d, sem_i)
        ci.wait()
        cj.wait()
        gets = [
            pltpu.async_copy(table_h.at[idx_s.at[j]], rows.at[j], sem_g)
            for j in range(CHUNK)
        ]
        for g in gets:
            g.wait()
        adds = [
            pltpu.async_copy(rows.at[j], acc.at[idx_d.at[j]], sem_s, add=True)
            for j in range(CHUNK)
        ]
        for a in adds:
            a.wait()

    plsc.subcore_barrier()
    _copy_out(acc, out_h, cid, sid)


def _sc_degree(dst3, zeros, ones):
    kern = pl.kernel(
        _deg_kernel,
        out_type=jax.ShapeDtypeStruct((NC, NPAD, F), jnp.float32),
        mesh=_sc_mesh(),
        scratch_types=[
            pltpu.VMEM_SHARED((NPAD, F), jnp.float32),
            pltpu.VMEM((CHUNK, LANE), jnp.int32),
            pltpu.VMEM((LANE, F), jnp.float32),
            pltpu.SemaphoreType.DMA,
            pltpu.SemaphoreType.DMA,
        ],
        compiler_params=_SC_PARAMS,
    )
    return kern(ones, dst3, zeros)


def _sc_aggregate(table, src3, dst3, zeros):
    kern = pl.kernel(
        _agg_kernel,
        out_type=jax.ShapeDtypeStruct((NC, NPAD, F), jnp.float32),
        mesh=_sc_mesh(),
        scratch_types=[
            pltpu.VMEM_SHARED((NPAD, F), jnp.float32),
            pltpu.VMEM((CHUNK, LANE), jnp.int32),
            pltpu.VMEM((CHUNK, LANE), jnp.int32),
            pltpu.VMEM((CHUNK, LANE, F), jnp.float32),
            pltpu.SemaphoreType.DMA,
            pltpu.SemaphoreType.DMA,
            pltpu.SemaphoreType.DMA,
        ],
        compiler_params=_SC_PARAMS,
    )
    return kern(table, src3, dst3, zeros)


def _ew_loop(body):
    @pl.loop(0, ROWS_W)
    def _(r):
        @pl.loop(0, LANE, step=16)
        def _(c):
            body(r, pl.ds(c, 16))


def _prescale_kernel(deg_h, x_h, y1_h, d4_h, deg0_v, deg1_v, x_v,
                     y1_v, d4_v, sem):
    _, _, wid = _worker_ids()
    sl = pl.ds(wid * ROWS_W, ROWS_W)
    c0 = pltpu.async_copy(deg_h.at[0].at[sl], deg0_v, sem)
    c1 = pltpu.async_copy(deg_h.at[1].at[sl], deg1_v, sem)
    c2 = pltpu.async_copy(x_h.at[sl], x_v, sem)
    c0.wait()
    c1.wait()
    c2.wait()

    def body(r, cs):
        deg = deg0_v[r, cs] + deg1_v[r, cs] + 1.0
        d = _rsqrt16(deg)
        d4_v[r, cs] = d
        y1_v[r, cs] = x_v[r, cs] * d

    _ew_loop(body)
    o0 = pltpu.async_copy(y1_v, y1_h.at[sl], sem)
    o1 = pltpu.async_copy(d4_v, d4_h.at[sl], sem)
    o0.wait()
    o1.wait()


def _sc_prescale(deg_parts_r, x_r):
    kern = pl.kernel(
        _prescale_kernel,
        out_type=[
            jax.ShapeDtypeStruct((PACK, LANE), jnp.float32),
            jax.ShapeDtypeStruct((PACK, LANE), jnp.float32),
        ],
        mesh=_sc_mesh(),
        scratch_types=[
            pltpu.VMEM((ROWS_W, LANE), jnp.float32),
            pltpu.VMEM((ROWS_W, LANE), jnp.float32),
            pltpu.VMEM((ROWS_W, LANE), jnp.float32),
            pltpu.VMEM((ROWS_W, LANE), jnp.float32),
            pltpu.VMEM((ROWS_W, LANE), jnp.float32),
            pltpu.SemaphoreType.DMA,
        ],
        compiler_params=_SC_PARAMS,
    )
    return kern(deg_parts_r, x_r)


def _midelem_kernel(a1_h, x_h, d4_h, pre_h, a0_v, a1_v, x_v, d4_v, pre_v, sem):
    _, _, wid = _worker_ids()
    sl = pl.ds(wid * ROWS_W, ROWS_W)
    cs = [pltpu.async_copy(a1_h.at[0].at[sl], a0_v, sem),
          pltpu.async_copy(a1_h.at[1].at[sl], a1_v, sem),
          pltpu.async_copy(x_h.at[sl], x_v, sem),
          pltpu.async_copy(d4_h.at[sl], d4_v, sem)]
    for c in cs:
        c.wait()

    def body(r, s):
        d = d4_v[r, s]
        pre_v[r, s] = (a0_v[r, s] + a1_v[r, s]) * d + x_v[r, s] * (d * d)

    _ew_loop(body)
    pltpu.async_copy(pre_v, pre_h.at[sl], sem).wait()


def _sc_midelem(a1_parts_r, x_r, d4):
    kern = pl.kernel(
        _midelem_kernel,
        out_type=jax.ShapeDtypeStruct((PACK, LANE), jnp.float32),
        mesh=_sc_mesh(),
        scratch_types=[
            pltpu.VMEM((ROWS_W, LANE), jnp.float32),
            pltpu.VMEM((ROWS_W, LANE), jnp.float32),
            pltpu.VMEM((ROWS_W, LANE), jnp.float32),
            pltpu.VMEM((ROWS_W, LANE), jnp.float32),
            pltpu.VMEM((ROWS_W, LANE), jnp.float32),
            pltpu.SemaphoreType.DMA,
        ],
        compiler_params=_SC_PARAMS,
    )
    return kern(a1_parts_r, x_r, d4)


def _final_kernel(a2_h, y2_h, d4_h, b2_h, out_h,
                  a0_v, a1_v, y2_v, d4_v, t_v, o_v, b2_v, sem):
    _, _, wid = _worker_ids()
    sl = pl.ds(wid * ROWS_W, ROWS_W)
    cs = [pltpu.async_copy(a2_h.at[0].at[sl], a0_v, sem),
          pltpu.async_copy(a2_h.at[1].at[sl], a1_v, sem),
          pltpu.async_copy(y2_h.at[sl], y2_v, sem),
          pltpu.async_copy(d4_h.at[sl], d4_v, sem),
          pltpu.async_copy(b2_h, b2_v, sem)]
    for c in cs:
        c.wait()

    def body(r, s):
        t_v[r, s] = d4_v[r, s] * (a0_v[r, s] + a1_v[r, s] + y2_v[r, s])

    _ew_loop(body)

    bias = b2_v[...]
    lanes = lax.iota(jnp.int32, 16) * 4

    @pl.loop(0, NODES_W, step=16)
    def _(n):
        r = n // 32
        c = (n % 32) * 4
        v = plsc.load_gather(t_v, [jnp.full((16,), r, jnp.int32),
                                   lanes + c])
        z = v + bias
        o_v[pl.ds(n, 16)] = 1.0 / (1.0 + jnp.exp(-z))

    pltpu.async_copy(o_v, out_h.at[pl.ds(wid * NODES_W, NODES_W)], sem).wait()


def _sc_final(a2_parts_r, y2_r, d4, b2x):
    kern = pl.kernel(
        _final_kernel,
        out_type=jax.ShapeDtypeStruct((NPAD,), jnp.float32),
        mesh=_sc_mesh(),
        scratch_types=[
            pltpu.VMEM((ROWS_W, LANE), jnp.float32),
            pltpu.VMEM((ROWS_W, LANE), jnp.float32),
            pltpu.VMEM((ROWS_W, LANE), jnp.float32),
            pltpu.VMEM((ROWS_W, LANE), jnp.float32),
            pltpu.VMEM((ROWS_W, LANE), jnp.float32),
            pltpu.VMEM((NODES_W,), jnp.float32),
            pltpu.VMEM((16,), jnp.float32),
            pltpu.SemaphoreType.DMA,
        ],
        compiler_params=_SC_EW_PARAMS,
    )
    return kern(a2_parts_r, y2_r, d4, b2x)


TC_BLK = 800


def _tc_mid_body(pre, d4, w1b, b1b, w2b, y2_out):
    h = jnp.dot(pre[...], w1b[...], preferred_element_type=jnp.float32)
    h = jnp.maximum(h + b1b[...], 0.0)
    s4 = jnp.dot(h, w2b[...], preferred_element_type=jnp.float32)
    y2_out[...] = s4 * d4[...]


def _tc_mid(pre, d4, W1, b1, W2):
    eye = jnp.eye(32, dtype=jnp.float32)
    w1b = jnp.kron(eye, W1)
    b1b = jnp.tile(b1, 32).reshape(1, 2048)
    w2b = jnp.kron(eye, jnp.broadcast_to(W2, (64, 4)))
    grid = PACK // TC_BLK
    blk = pl.BlockSpec((TC_BLK, LANE), lambda i: (i, 0))
    return pl.pallas_call(
        _tc_mid_body,
        grid=(grid,),
        in_specs=[blk, blk,
                  pl.BlockSpec((128, 2048), lambda i: (0, 0)),
                  pl.BlockSpec((1, 2048), lambda i: (0, 0)),
                  pl.BlockSpec((2048, 128), lambda i: (0, 0))],
        out_specs=blk,
        out_shape=jax.ShapeDtypeStruct((PACK, LANE), jnp.float32),
    )(pre, d4, w1b, b1b, w2b)


DEBUG_CUT = None
USE_SC_PRESCALE = True
USE_SC_MID = False
USE_SC_FINAL = False


@jax.jit
def kernel(x, edge_index, W1, b1, W2, b2):
    if DEBUG_CUT is not None:
        return _kernel_hybrid(x, edge_index, W1, b1, W2, b2)
    return _kernel_full(x, edge_index, W1, b1, W2, b2)


def _kernel_hybrid(x, edge_index, W1, b1, W2, b2):
    n = x.shape[0]
    e = edge_index.shape[1]
    e32 = edge_index.astype(jnp.int32)
    fill = jnp.full((EPAD - e,), n, dtype=jnp.int32)
    src3 = jnp.concatenate([e32[0], fill]).reshape(NW, RPW, LANE)
    dst3 = jnp.concatenate([e32[1], fill]).reshape(NW, RPW, LANE)
    x_pad = jnp.zeros((NPAD, F), jnp.float32).at[:n].set(x)
    x_r = x_pad.reshape(PACK, LANE)
    zeros = jnp.zeros((NPAD, F), jnp.float32)
    ones = jnp.ones((LANE, F), jnp.float32)
    b2x = jnp.broadcast_to(b2, (16,)).astype(jnp.float32)

    if DEBUG_CUT >= 1:
        deg_parts = _sc_degree(dst3, zeros, ones)
        deg = deg_parts[0][:, 0:1] + deg_parts[1][:, 0:1] + 1.0
    else:
        deg_edges = jnp.zeros((NPAD,), jnp.float32).at[e32[1]].add(1.0)
        deg = deg_edges[:, None] + 1.0
        deg_parts = jnp.stack([deg_edges[:, None] * jnp.ones((1, F)),
                               jnp.zeros((NPAD, F))])
    if DEBUG_CUT >= 2:
        y1r, d4r = _sc_prescale(deg_parts.reshape(NC, PACK, LANE), x_r)
        d_full = d4r.reshape(NPAD, F)
        y1 = y1r.reshape(NPAD, F)
    else:
        d_full = jnp.broadcast_to(lax.rsqrt(deg), (NPAD, F))
        y1 = x_pad * d_full
    if DEBUG_CUT >= 3:
        a1_parts = _sc_aggregate(y1, src3, dst3, zeros)
        a1 = a1_parts[0] + a1_parts[1]
    else:
        a1 = jnp.zeros((NPAD, F), jnp.float32).at[e32[1]].add(y1[e32[0]])
    pre = a1 * d_full + x_pad * d_full * d_full
    if DEBUG_CUT >= 5:
        pre_r = _sc_midelem(a1_parts.reshape(NC, PACK, LANE), x_r,
                            d_full.reshape(PACK, LANE))
        pre = pre_r.reshape(NPAD, F)
    h1 = jnp.maximum(pre @ W1 + b1, 0.0)
    s = h1 @ W2
    y2 = jnp.broadcast_to(s, (NPAD, F)) * d_full
    if DEBUG_CUT >= 4:
        a2_parts = _sc_aggregate(y2, src3, dst3, zeros)
        a2 = a2_parts[0] + a2_parts[1]
    else:
        a2 = jnp.zeros((NPAD, F), jnp.float32).at[e32[1]].add(y2[e32[0]])
    out = jax.nn.sigmoid(d_full[:, 0:1] * (a2[:, 0:1] + y2[:, 0:1]) + b2)
    return out[:n]


def _kernel_full(x, edge_index, W1, b1, W2, b2):
    n = x.shape[0]
    e = edge_index.shape[1]

    e32 = edge_index.astype(jnp.int32)
    fill = jnp.full((EPAD - e,), n, dtype=jnp.int32)
    src3 = jnp.concatenate([e32[0], fill]).reshape(NW, RPW, LANE)
    dst3 = jnp.concatenate([e32[1], fill]).reshape(NW, RPW, LANE)

    x_pad = jnp.zeros((NPAD, F), jnp.float32).at[:n].set(x)
    x_r = x_pad.reshape(PACK, LANE)
    zeros = jnp.zeros((NPAD, F), jnp.float32)
    ones = jnp.ones((LANE, F), jnp.float32)

    deg_parts = _sc_degree(dst3, zeros, ones)
    degp_r = deg_parts.reshape(NC, PACK, LANE)
    d4, y1r = _tc_prescale_pk(degp_r[0], degp_r[1], x_r)

    a1_parts = _sc_aggregate(y1r.reshape(NPAD, F), src3, dst3, zeros)
    a1p_r = a1_parts.reshape(NC, PACK, LANE)
    y2r = _tc_mid_pk(a1p_r[0], a1p_r[1], x_r, d4, W1, b1, W2)

    a2_parts = _sc_aggregate(y2r.reshape(NPAD, F), src3, dst3, zeros)
    a2p_r = a2_parts.reshape(NC, PACK, LANE)
    out4 = _tc_final_pk(a2p_r[0], a2p_r[1], y2r, d4, b2)
    return out4.reshape(NPAD, F)[:n, 0:1]


TC_BLKR = 800


def _tc_prescale_pk_body(deg0, deg1, x, d4_out, y1_out):
    d = lax.rsqrt(deg0[...] + deg1[...] + 1.0)
    d4_out[...] = d
    y1_out[...] = x[...] * d


def _tc_prescale_pk(deg0_r, deg1_r, x_r):
    grid = PACK // TC_BLKR
    blk = pl.BlockSpec((TC_BLKR, LANE), lambda i: (i, 0))
    return pl.pallas_call(
        _tc_prescale_pk_body,
        grid=(grid,),
        in_specs=[blk, blk, blk],
        out_specs=[blk, blk],
        out_shape=[
            jax.ShapeDtypeStruct((PACK, LANE), jnp.float32),
            jax.ShapeDtypeStruct((PACK, LANE), jnp.float32),
        ],
    )(deg0_r, deg1_r, x_r)


def _tc_mid_pk_body(a0, a1, x, d4, w1b, b1b, w2b, y2_out):
    dv = d4[...]
    pre = (a0[...] + a1[...]) * dv + x[...] * (dv * dv)
    h = jnp.dot(pre, w1b[...], preferred_element_type=jnp.float32)
    h = jnp.maximum(h + b1b[...], 0.0)
    s4 = jnp.dot(h, w2b[...], preferred_element_type=jnp.float32)
    y2_out[...] = s4 * dv


def _tc_mid_pk(a0_r, a1_r, x_r, d4, W1, b1, W2):
    eye = jnp.eye(32, dtype=jnp.float32)
    w1b = jnp.kron(eye, W1)
    b1b = jnp.tile(b1, 32).reshape(1, 2048)
    w2b = jnp.kron(eye, jnp.broadcast_to(W2, (64, 4)))
    grid = PACK // TC_BLKR
    blk = pl.BlockSpec((TC_BLKR, LANE), lambda i: (i, 0))
    return pl.pallas_call(
        _tc_mid_pk_body,
        grid=(grid,),
        in_specs=[blk, blk, blk, blk,
                  pl.BlockSpec((128, 2048), lambda i: (0, 0)),
                  pl.BlockSpec((1, 2048), lambda i: (0, 0)),
                  pl.BlockSpec((2048, 128), lambda i: (0, 0))],
        out_specs=blk,
        out_shape=jax.ShapeDtypeStruct((PACK, LANE), jnp.float32),
    )(a0_r, a1_r, x_r, d4, w1b, b1b, w2b)


def _tc_final_pk_body(a0, a1, y2, d4, b2, out):
    t = d4[...] * (a0[...] + a1[...] + y2[...]) + b2[...]
    out[...] = jax.nn.sigmoid(t)


def _tc_final_pk(a0_r, a1_r, y2r, d4, b2):
    grid = PACK // TC_BLKR
    blk = pl.BlockSpec((TC_BLKR, LANE), lambda i: (i, 0))
    return pl.pallas_call(
        _tc_final_pk_body,
        grid=(grid,),
        in_specs=[blk, blk, blk, blk,
                  pl.BlockSpec((1, 1), lambda i: (0, 0))],
        out_specs=blk,
        out_shape=jax.ShapeDtypeStruct((PACK, LANE), jnp.float32),
    )(a0_r, a1_r, y2r, d4, b2.reshape(1, 1))


# speedup vs baseline: 1.2072x; 1.2072x over previous
import jax
import jax.numpy as jnp
from jax import lax
from jax.experimental import pallas as pl
from jax.experimental.pallas import tpu as pltpu
from jax.experimental.pallas import tpu_sc as plsc

NC = 2
NS = 16
NW = NC * NS
LANE = 128
CHUNK = 8
F = 4

NPAD = 102400
RPW = 784
EPAD = NW * RPW * LANE
SLICE = NPAD // NS
NODES_W = NPAD // NW
ROWS_W = NODES_W * F // LANE
PACK = NPAD * F // LANE


def _sc_mesh():
    return plsc.VectorSubcoreMesh(core_axis_name="c", subcore_axis_name="s")


_SC_PARAMS = pltpu.CompilerParams(use_tc_tiling_on_sc=False)
_SC_EW_PARAMS = pltpu.CompilerParams(use_tc_tiling_on_sc=False,
                                     needs_layout_passes=False)


def _worker_ids():
    cid = lax.axis_index("c")
    sid = lax.axis_index("s")
    return cid, sid, sid * NC + cid


def _zero_acc(zeros_h, acc, sid):
    sl = pl.ds(sid * SLICE, SLICE)
    pltpu.sync_copy(zeros_h.at[sl], acc.at[sl])


def _copy_out(acc, out_h, cid, sid):
    sl = pl.ds(sid * SLICE, SLICE)
    pltpu.sync_copy(acc.at[sl], out_h.at[cid].at[sl])


def _rsqrt16(v):
    i = plsc.bitcast(v, jnp.int32)
    i = jnp.int32(0x5F3759DF) - lax.shift_right_arithmetic(i, 1)
    y = plsc.bitcast(i, jnp.float32)
    for _ in range(3):
        y = y * (1.5 - 0.5 * v * y * y)
    return y


def _deg_kernel(ones_h, dst_h, zeros_h, out_h, acc, idx_d, ones_v, sem_i, sem_s):
    cid, sid, wid = _worker_ids()
    _zero_acc(zeros_h, acc, sid)
    pltpu.sync_copy(ones_h, ones_v)
    plsc.subcore_barrier()

    @pl.loop(0, RPW, step=CHUNK)
    def _(i):
        pltpu.async_copy(dst_h.at[wid].at[pl.ds(i, CHUNK)], idx_d, sem_i).wait()
        adds = [
            pltpu.async_copy(ones_v, acc.at[idx_d.at[j]], sem_s, add=True)
            for j in range(CHUNK)
        ]
        for a in adds:
            a.wait()

    plsc.subcore_barrier()
    _copy_out(acc, out_h, cid, sid)


def _agg_kernel(table_h, src_h, dst_h, zeros_h, out_h,
                acc, idx_s, idx_d, rows, sem_i, sem_g, sem_s):
    cid, sid, wid = _worker_ids()
    _zero_acc(zeros_h, acc, sid)
    plsc.subcore_barrier()

    @pl.loop(0, RPW, step=CHUNK)
    def _(i):
        ci = pltpu.async_copy(src_h.at[wid].at[pl.ds(i, CHUNK)], idx_s, sem_i)
        cj = pltpu.async_copy(dst_h.at[wid].at[pl.ds(i, CHUNK)], idx_d, sem_i)
        ci.wait()
        cj.wait()
        gets = [
            pltpu.async_copy(table_h.at[idx_s.at[j]], rows.at[j], sem_g)
            for j in range(CHUNK)
        ]
        for g in gets:
            g.wait()
        adds = [
            pltpu.async_copy(rows.at[j], acc.at[idx_d.at[j]], sem_s, add=True)
            for j in range(CHUNK)
        ]
        for a in adds:
            a.wait()

    plsc.subcore_barrier()
    _copy_out(acc, out_h, cid, sid)


def _sc_degree(dst3, zeros, ones):
    kern = pl.kernel(
        _deg_kernel,
        out_type=jax.ShapeDtypeStruct((NC, NPAD, F), jnp.float32),
        mesh=_sc_mesh(),
        scratch_types=[
            pltpu.VMEM_SHARED((NPAD, F), jnp.float32),
            pltpu.VMEM((CHUNK, LANE), jnp.int32),
            pltpu.VMEM((LANE, F), jnp.float32),
            pltpu.SemaphoreType.DMA,
            pltpu.SemaphoreType.DMA,
        ],
        compiler_params=_SC_PARAMS,
    )
    return kern(ones, dst3, zeros)


def _sc_aggregate(table, src3, dst3, zeros):
    kern = pl.kernel(
        _agg_kernel,
        out_type=jax.ShapeDtypeStruct((NC, NPAD, F), jnp.float32),
        mesh=_sc_mesh(),
        scratch_types=[
            pltpu.VMEM_SHARED((NPAD, F), jnp.float32),
            pltpu.VMEM((CHUNK, LANE), jnp.int32),
            pltpu.VMEM((CHUNK, LANE), jnp.int32),
            pltpu.VMEM((CHUNK, LANE, F), jnp.float32),
            pltpu.SemaphoreType.DMA,
            pltpu.SemaphoreType.DMA,
            pltpu.SemaphoreType.DMA,
        ],
        compiler_params=_SC_PARAMS,
    )
    return kern(table, src3, dst3, zeros)


def _ew_loop(body):
    @pl.loop(0, ROWS_W)
    def _(r):
        @pl.loop(0, LANE, step=16)
        def _(c):
            body(r, pl.ds(c, 16))


def _prescale_kernel(deg_h, x_h, y1_h, d4_h, deg0_v, deg1_v, x_v,
                     y1_v, d4_v, sem):
    _, _, wid = _worker_ids()
    sl = pl.ds(wid * ROWS_W, ROWS_W)
    c0 = pltpu.async_copy(deg_h.at[0].at[sl], deg0_v, sem)
    c1 = pltpu.async_copy(deg_h.at[1].at[sl], deg1_v, sem)
    c2 = pltpu.async_copy(x_h.at[sl], x_v, sem)
    c0.wait()
    c1.wait()
    c2.wait()

    def body(r, cs):
        deg = deg0_v[r, cs] + deg1_v[r, cs] + 1.0
        d = _rsqrt16(deg)
        d4_v[r, cs] = d
        y1_v[r, cs] = x_v[r, cs] * d

    _ew_loop(body)
    o0 = pltpu.async_copy(y1_v, y1_h.at[sl], sem)
    o1 = pltpu.async_copy(d4_v, d4_h.at[sl], sem)
    o0.wait()
    o1.wait()


def _sc_prescale(deg_parts_r, x_r):
    kern = pl.kernel(
        _prescale_kernel,
        out_type=[
            jax.ShapeDtypeStruct((PACK, LANE), jnp.float32),
            jax.ShapeDtypeStruct((PACK, LANE), jnp.float32),
        ],
        mesh=_sc_mesh(),
        scratch_types=[
            pltpu.VMEM((ROWS_W, LANE), jnp.float32),
            pltpu.VMEM((ROWS_W, LANE), jnp.float32),
            pltpu.VMEM((ROWS_W, LANE), jnp.float32),
            pltpu.VMEM((ROWS_W, LANE), jnp.float32),
            pltpu.VMEM((ROWS_W, LANE), jnp.float32),
            pltpu.SemaphoreType.DMA,
        ],
        compiler_params=_SC_PARAMS,
    )
    return kern(deg_parts_r, x_r)


def _midelem_kernel(a1_h, x_h, d4_h, pre_h, a0_v, a1_v, x_v, d4_v, pre_v, sem):
    _, _, wid = _worker_ids()
    sl = pl.ds(wid * ROWS_W, ROWS_W)
    cs = [pltpu.async_copy(a1_h.at[0].at[sl], a0_v, sem),
          pltpu.async_copy(a1_h.at[1].at[sl], a1_v, sem),
          pltpu.async_copy(x_h.at[sl], x_v, sem),
          pltpu.async_copy(d4_h.at[sl], d4_v, sem)]
    for c in cs:
        c.wait()

    def body(r, s):
        d = d4_v[r, s]
        pre_v[r, s] = (a0_v[r, s] + a1_v[r, s]) * d + x_v[r, s] * (d * d)

    _ew_loop(body)
    pltpu.async_copy(pre_v, pre_h.at[sl], sem).wait()


def _sc_midelem(a1_parts_r, x_r, d4):
    kern = pl.kernel(
        _midelem_kernel,
        out_type=jax.ShapeDtypeStruct((PACK, LANE), jnp.float32),
        mesh=_sc_mesh(),
        scratch_types=[
            pltpu.VMEM((ROWS_W, LANE), jnp.float32),
            pltpu.VMEM((ROWS_W, LANE), jnp.float32),
            pltpu.VMEM((ROWS_W, LANE), jnp.float32),
            pltpu.VMEM((ROWS_W, LANE), jnp.float32),
            pltpu.VMEM((ROWS_W, LANE), jnp.float32),
            pltpu.SemaphoreType.DMA,
        ],
        compiler_params=_SC_PARAMS,
    )
    return kern(a1_parts_r, x_r, d4)


def _final_kernel(a2_h, y2_h, d4_h, b2_h, out_h,
                  a0_v, a1_v, y2_v, d4_v, t_v, o_v, b2_v, sem):
    _, _, wid = _worker_ids()
    sl = pl.ds(wid * ROWS_W, ROWS_W)
    cs = [pltpu.async_copy(a2_h.at[0].at[sl], a0_v, sem),
          pltpu.async_copy(a2_h.at[1].at[sl], a1_v, sem),
          pltpu.async_copy(y2_h.at[sl], y2_v, sem),
          pltpu.async_copy(d4_h.at[sl], d4_v, sem),
          pltpu.async_copy(b2_h, b2_v, sem)]
    for c in cs:
        c.wait()

    def body(r, s):
        t_v[r, s] = d4_v[r, s] * (a0_v[r, s] + a1_v[r, s] + y2_v[r, s])

    _ew_loop(body)

    bias = b2_v[...]
    lanes = lax.iota(jnp.int32, 16) * 4

    @pl.loop(0, NODES_W, step=16)
    def _(n):
        r = n // 32
        c = (n % 32) * 4
        v = plsc.load_gather(t_v, [jnp.full((16,), r, jnp.int32),
                                   lanes + c])
        z = v + bias
        o_v[pl.ds(n, 16)] = 1.0 / (1.0 + jnp.exp(-z))

    pltpu.async_copy(o_v, out_h.at[pl.ds(wid * NODES_W, NODES_W)], sem).wait()


def _sc_final(a2_parts_r, y2_r, d4, b2x):
    kern = pl.kernel(
        _final_kernel,
        out_type=jax.ShapeDtypeStruct((NPAD,), jnp.float32),
        mesh=_sc_mesh(),
        scratch_types=[
            pltpu.VMEM((ROWS_W, LANE), jnp.float32),
            pltpu.VMEM((ROWS_W, LANE), jnp.float32),
            pltpu.VMEM((ROWS_W, LANE), jnp.float32),
            pltpu.VMEM((ROWS_W, LANE), jnp.float32),
            pltpu.VMEM((ROWS_W, LANE), jnp.float32),
            pltpu.VMEM((NODES_W,), jnp.float32),
            pltpu.VMEM((16,), jnp.float32),
            pltpu.SemaphoreType.DMA,
        ],
        compiler_params=_SC_EW_PARAMS,
    )
    return kern(a2_parts_r, y2_r, d4, b2x)


TC_BLK = 800


def _tc_mid_body(pre, d4, w1b, b1b, w2b, y2_out):
    h = jnp.dot(pre[...], w1b[...], preferred_element_type=jnp.float32)
    h = jnp.maximum(h + b1b[...], 0.0)
    s4 = jnp.dot(h, w2b[...], preferred_element_type=jnp.float32)
    y2_out[...] = s4 * d4[...]


def _tc_mid(pre, d4, W1, b1, W2):
    eye = jnp.eye(32, dtype=jnp.float32)
    w1b = jnp.kron(eye, W1)
    b1b = jnp.tile(b1, 32).reshape(1, 2048)
    w2b = jnp.kron(eye, jnp.broadcast_to(W2, (64, 4)))
    grid = PACK // TC_BLK
    blk = pl.BlockSpec((TC_BLK, LANE), lambda i: (i, 0))
    return pl.pallas_call(
        _tc_mid_body,
        grid=(grid,),
        in_specs=[blk, blk,
                  pl.BlockSpec((128, 2048), lambda i: (0, 0)),
                  pl.BlockSpec((1, 2048), lambda i: (0, 0)),
                  pl.BlockSpec((2048, 128), lambda i: (0, 0))],
        out_specs=blk,
        out_shape=jax.ShapeDtypeStruct((PACK, LANE), jnp.float32),
    )(pre, d4, w1b, b1b, w2b)


DEBUG_CUT = None
USE_SC_PRESCALE = True
USE_SC_MID = False
USE_SC_FINAL = False


@jax.jit
def kernel(x, edge_index, W1, b1, W2, b2):
    if DEBUG_CUT is not None:
        return _kernel_hybrid(x, edge_index, W1, b1, W2, b2)
    return _kernel_full(x, edge_index, W1, b1, W2, b2)


def _kernel_hybrid(x, edge_index, W1, b1, W2, b2):
    n = x.shape[0]
    e = edge_index.shape[1]
    e32 = edge_index.astype(jnp.int32)
    fill = jnp.full((EPAD - e,), n, dtype=jnp.int32)
    src3 = jnp.concatenate([e32[0], fill]).reshape(NW, RPW, LANE)
    dst3 = jnp.concatenate([e32[1], fill]).reshape(NW, RPW, LANE)
    x_pad = jnp.zeros((NPAD, F), jnp.float32).at[:n].set(x)
    x_r = x_pad.reshape(PACK, LANE)
    zeros = jnp.zeros((NPAD, F), jnp.float32)
    ones = jnp.ones((LANE, F), jnp.float32)
    b2x = jnp.broadcast_to(b2, (16,)).astype(jnp.float32)

    if DEBUG_CUT >= 1:
        deg_parts = _sc_degree(dst3, zeros, ones)
        deg = deg_parts[0][:, 0:1] + deg_parts[1][:, 0:1] + 1.0
    else:
        deg_edges = jnp.zeros((NPAD,), jnp.float32).at[e32[1]].add(1.0)
        deg = deg_edges[:, None] + 1.0
        deg_parts = jnp.stack([deg_edges[:, None] * jnp.ones((1, F)),
                               jnp.zeros((NPAD, F))])
    if DEBUG_CUT >= 2:
        y1r, d4r = _sc_prescale(deg_parts.reshape(NC, PACK, LANE), x_r)
        d_full = d4r.reshape(NPAD, F)
        y1 = y1r.reshape(NPAD, F)
    else:
        d_full = jnp.broadcast_to(lax.rsqrt(deg), (NPAD, F))
        y1 = x_pad * d_full
    if DEBUG_CUT >= 3:
        a1_parts = _sc_aggregate(y1, src3, dst3, zeros)
        a1 = a1_parts[0] + a1_parts[1]
    else:
        a1 = jnp.zeros((NPAD, F), jnp.float32).at[e32[1]].add(y1[e32[0]])
    pre = a1 * d_full + x_pad * d_full * d_full
    if DEBUG_CUT >= 5:
        pre_r = _sc_midelem(a1_parts.reshape(NC, PACK, LANE), x_r,
                            d_full.reshape(PACK, LANE))
        pre = pre_r.reshape(NPAD, F)
    h1 = jnp.maximum(pre @ W1 + b1, 0.0)
    s = h1 @ W2
    y2 = jnp.broadcast_to(s, (NPAD, F)) * d_full
    if DEBUG_CUT >= 4:
        a2_parts = _sc_aggregate(y2, src3, dst3, zeros)
        a2 = a2_parts[0] + a2_parts[1]
    else:
        a2 = jnp.zeros((NPAD, F), jnp.float32).at[e32[1]].add(y2[e32[0]])
    out = jax.nn.sigmoid(d_full[:, 0:1] * (a2[:, 0:1] + y2[:, 0:1]) + b2)
    return out[:n]


def _kernel_full(x, edge_index, W1, b1, W2, b2):
    n = x.shape[0]
    e = edge_index.shape[1]

    e32 = edge_index.astype(jnp.int32)
    fill = jnp.full((EPAD - e,), n, dtype=jnp.int32)
    src3 = jnp.concatenate([e32[0], fill]).reshape(NW, RPW, LANE)
    dst3 = jnp.concatenate([e32[1], fill]).reshape(NW, RPW, LANE)

    x_pad = jnp.zeros((NPAD, F), jnp.float32).at[:n].set(x)
    x_r = x_pad.reshape(PACK, LANE)
    zeros = jnp.zeros((NPAD, F), jnp.float32)
    ones = jnp.ones((LANE, F), jnp.float32)

    degp_r = _sc_degree(dst3, zeros, ones).reshape(NC, PACK, LANE)
    d4, y1r = _tc_prescale_pk(degp_r, x_r)

    a1p = _sc_aggregate(y1r.reshape(NPAD, F), src3, dst3, zeros)
    y2r = _tc_mid_pk(a1p.reshape(NC, PACK, LANE), x_r, d4, W1, b1, W2)

    a2p = _sc_aggregate(y2r.reshape(NPAD, F), src3, dst3, zeros)
    outp = _tc_final_pk(a2p.reshape(NC, PACK, LANE), y2r, d4, b2)
    return outp.reshape(NPAD, 1)[:n]


TC_BLKR = 800


def _tc_prescale_pk_body(degp, x, d4_out, y1_out):
    d = lax.rsqrt(degp[0] + degp[1] + 1.0)
    d4_out[...] = d
    y1_out[...] = x[...] * d


def _tc_prescale_pk(degp_r, x_r):
    grid = PACK // TC_BLKR
    blk = pl.BlockSpec((TC_BLKR, LANE), lambda i: (i, 0))
    blk2 = pl.BlockSpec((NC, TC_BLKR, LANE), lambda i: (0, i, 0))
    return pl.pallas_call(
        _tc_prescale_pk_body,
        grid=(grid,),
        in_specs=[blk2, blk],
        out_specs=[blk, blk],
        out_shape=[
            jax.ShapeDtypeStruct((PACK, LANE), jnp.float32),
            jax.ShapeDtypeStruct((PACK, LANE), jnp.float32),
        ],
    )(degp_r, x_r)


def _tc_mid_pk_body(ap, x, d4, w1b, b1b, w2b, y2_out):
    dv = d4[...]
    pre = (ap[0] + ap[1]) * dv + x[...] * (dv * dv)
    h = jnp.dot(pre, w1b[...], preferred_element_type=jnp.float32)
    h = jnp.maximum(h + b1b[...], 0.0)
    s4 = jnp.dot(h, w2b[...], preferred_element_type=jnp.float32)
    y2_out[...] = s4 * dv


def _tc_mid_pk(ap_r, x_r, d4, W1, b1, W2):
    eye = jnp.eye(32, dtype=jnp.float32)
    w1b = jnp.kron(eye, W1)
    b1b = jnp.tile(b1, 32).reshape(1, 2048)
    w2b = jnp.kron(eye, jnp.broadcast_to(W2, (64, 4)))
    grid = PACK // TC_BLKR
    blk = pl.BlockSpec((TC_BLKR, LANE), lambda i: (i, 0))
    blk2 = pl.BlockSpec((NC, TC_BLKR, LANE), lambda i: (0, i, 0))
    return pl.pallas_call(
        _tc_mid_pk_body,
        grid=(grid,),
        in_specs=[blk2, blk, blk,
                  pl.BlockSpec((128, 2048), lambda i: (0, 0)),
                  pl.BlockSpec((1, 2048), lambda i: (0, 0)),
                  pl.BlockSpec((2048, 128), lambda i: (0, 0))],
        out_specs=blk,
        out_shape=jax.ShapeDtypeStruct((PACK, LANE), jnp.float32),
    )(ap_r, x_r, d4, w1b, b1b, w2b)


def _tc_final_pk_body(ap, y2, d4, b2, sel, out):
    t = d4[...] * (ap[0] + ap[1] + y2[...]) + b2[...]
    sg = jax.nn.sigmoid(t)
    out[...] = jnp.dot(sg, sel[...], preferred_element_type=jnp.float32)


def _tc_final_pk(ap_r, y2r, d4, b2):
    lanes = jnp.arange(32)
    sel = jnp.zeros((128, 32), jnp.float32).at[lanes * 4, lanes].set(1.0)
    grid = PACK // TC_BLKR
    blk = pl.BlockSpec((TC_BLKR, LANE), lambda i: (i, 0))
    blk2 = pl.BlockSpec((NC, TC_BLKR, LANE), lambda i: (0, i, 0))
    blko = pl.BlockSpec((TC_BLKR, 32), lambda i: (i, 0))
    return pl.pallas_call(
        _tc_final_pk_body,
        grid=(grid,),
        in_specs=[blk2, blk, blk,
                  pl.BlockSpec((1, 1), lambda i: (0, 0)),
                  pl.BlockSpec((128, 32), lambda i: (0, 0))],
        out_specs=blko,
        out_shape=jax.ShapeDtypeStruct((PACK, 32), jnp.float32),
    )(ap_r, y2r, d4, b2.reshape(1, 1), sel)
